# baseline (speedup 1.0000x reference)
import jax
import jax.numpy as jnp
from jax import lax
from jax.experimental import pallas as pl
from jax.experimental.pallas import tpu as pltpu
from jax.experimental.pallas import tpu_sc as plsc

N = 10000
D = 128
DH = D // 2
E = 320000
NC = 2
NS = 16
K = 128
CT = 160
EPAD = NS * CT * K
NCH_PH = 40
NPH = CT // NCH_PH
JUNK = 512
ACC_ROWS = N + JUNK
IOSUB = 10
IOROWS = N // IOSUB


def _sc_body(feath_hbm, src_hbm, dst_hbm, out_hbm,
             feats, acc, idx_s, idx_d, rows0, rows1, sem0, sem1):
    c = lax.axis_index("c")
    s = lax.axis_index("s")

    io_base = pl.multiple_of(s * IOROWS, 8)

    @pl.when(s < IOSUB)
    def _init():
        pltpu.sync_copy(feath_hbm.at[c, pl.ds(io_base, IOROWS)],
                        feats.at[pl.ds(io_base, IOROWS)])
        pltpu.sync_copy(feath_hbm.at[c, pl.ds(io_base, IOROWS)],
                        acc.at[pl.ds(io_base, IOROWS)])

    plsc.subcore_barrier()

    for p in range(NPH):
        slab = pl.multiple_of(s * CT + p * NCH_PH, 8)
        pltpu.sync_copy(src_hbm.at[pl.ds(slab, NCH_PH)], idx_s)
        pltpu.sync_copy(dst_hbm.at[pl.ds(slab, NCH_PH)], idx_d)

        pltpu.async_copy(feats.at[idx_s.at[0]], rows0, sem0)

        def body(i, carry):
            j0 = 2 * i
            j1 = j0 + 1
            pltpu.async_copy(feats.at[idx_s.at[j1]], rows1, sem1)
            pltpu.make_async_copy(feats.at[idx_s.at[j0]], rows0, sem0).wait()
            pltpu.sync_copy(rows0, acc.at[idx_d.at[j0]], add=True)

            @pl.when(j1 + 1 < NCH_PH)
            def _():
                pltpu.async_copy(feats.at[idx_s.at[j1 + 1]], rows0, sem0)

            pltpu.make_async_copy(feats.at[idx_s.at[j1]], rows1, sem1).wait()
            pltpu.sync_copy(rows1, acc.at[idx_d.at[j1]], add=True)
            return carry

        lax.fori_loop(0, NCH_PH // 2, body, 0)

    plsc.subcore_barrier()

    @pl.when(s < IOSUB)
    def _writeout():
        pltpu.sync_copy(acc.at[pl.ds(io_base, IOROWS)],
                        out_hbm.at[c, pl.ds(io_base, IOROWS)])


_sc_gin = pl.kernel(
    _sc_body,
    out_type=jax.ShapeDtypeStruct((NC, N, DH), jnp.float32),
    mesh=plsc.VectorSubcoreMesh(
        core_axis_name="c", subcore_axis_name="s", num_cores=NC, num_subcores=NS
    ),
    compiler_params=pltpu.CompilerParams(use_tc_tiling_on_sc=False),
    scratch_types=[
        pltpu.VMEM_SHARED((N, DH), jnp.float32),
        pltpu.VMEM_SHARED((ACC_ROWS, DH), jnp.float32),
        pltpu.VMEM((NCH_PH, K), jnp.int32),
        pltpu.VMEM((NCH_PH, K), jnp.int32),
        pltpu.VMEM((K, DH), jnp.float32),
        pltpu.VMEM((K, DH), jnp.float32),
        pltpu.SemaphoreType.DMA,
        pltpu.SemaphoreType.DMA,
    ],
)


@jax.jit
def kernel(feat, edge_index):
    ei = edge_index.astype(jnp.int32)
    npad = EPAD - E
    src = jnp.concatenate([ei[0], jnp.zeros((npad,), jnp.int32)])
    pad_dst = N + (jnp.arange(npad, dtype=jnp.int32) % JUNK)
    dst = jnp.concatenate([ei[1], pad_dst])
    src = src.reshape(NS * CT, K)
    dst = dst.reshape(NS * CT, K)
    feat_halves = jnp.moveaxis(feat.reshape(N, NC, DH), 1, 0)
    out_halves = _sc_gin(feat_halves, src, dst)
    return jnp.moveaxis(out_halves, 0, 1).reshape(N, D)


# speedup vs baseline: 1.1894x; 1.1894x over previous
import jax
import jax.numpy as jnp
from jax import lax
from jax.experimental import pallas as pl
from jax.experimental.pallas import tpu as pltpu
from jax.experimental.pallas import tpu_sc as plsc

N = 10000
D = 128
DH = D // 2
E = 320000
NC = 2
NS = 16
K = 128
CT = 160
EPAD = NS * CT * K
NCH_PH = 40
NPH = CT // NCH_PH
JUNK = 512
ACC_ROWS = N + JUNK
IOSUB = 10
IOROWS = N // IOSUB


def _sc_body(feat_hbm, src_hbm, dst_hbm, out_hbm,
             feats, acc, idx_s, idx_d, rows0, rows1, sem0, sem1):
    c = lax.axis_index("c")
    s = lax.axis_index("s")

    io_base = pl.multiple_of(s * IOROWS, 8)
    col = pl.multiple_of(c * DH, DH)

    @pl.when(s < IOSUB)
    def _init():
        pltpu.sync_copy(feat_hbm.at[pl.ds(io_base, IOROWS), pl.ds(col, DH)],
                        feats.at[pl.ds(io_base, IOROWS)])
        pltpu.sync_copy(feat_hbm.at[pl.ds(io_base, IOROWS), pl.ds(col, DH)],
                        acc.at[pl.ds(io_base, IOROWS)])

    plsc.subcore_barrier()

    for p in range(NPH):
        slab = pl.multiple_of(s * CT + p * NCH_PH, 8)
        pltpu.sync_copy(src_hbm.at[pl.ds(slab, NCH_PH)], idx_s)
        pltpu.sync_copy(dst_hbm.at[pl.ds(slab, NCH_PH)], idx_d)

        pltpu.async_copy(feats.at[idx_s.at[0]], rows0, sem0)

        def body(i, carry):
            j0 = 2 * i
            j1 = j0 + 1
            pltpu.async_copy(feats.at[idx_s.at[j1]], rows1, sem1)
            pltpu.make_async_copy(feats.at[idx_s.at[j0]], rows0, sem0).wait()
            pltpu.sync_copy(rows0, acc.at[idx_d.at[j0]], add=True)

            @pl.when(j1 + 1 < NCH_PH)
            def _():
                pltpu.async_copy(feats.at[idx_s.at[j1 + 1]], rows0, sem0)

            pltpu.make_async_copy(feats.at[idx_s.at[j1]], rows1, sem1).wait()
            pltpu.sync_copy(rows1, acc.at[idx_d.at[j1]], add=True)
            return carry

        lax.fori_loop(0, NCH_PH // 2, body, 0)

    plsc.subcore_barrier()

    @pl.when(s < IOSUB)
    def _writeout():
        pltpu.sync_copy(acc.at[pl.ds(io_base, IOROWS)],
                        out_hbm.at[pl.ds(io_base, IOROWS), pl.ds(col, DH)])


_sc_gin = pl.kernel(
    _sc_body,
    out_type=jax.ShapeDtypeStruct((N, D), jnp.float32),
    mesh=plsc.VectorSubcoreMesh(
        core_axis_name="c", subcore_axis_name="s", num_cores=NC, num_subcores=NS
    ),
    compiler_params=pltpu.CompilerParams(use_tc_tiling_on_sc=False),
    scratch_types=[
        pltpu.VMEM_SHARED((N, DH), jnp.float32),
        pltpu.VMEM_SHARED((ACC_ROWS, DH), jnp.float32),
        pltpu.VMEM((NCH_PH, K), jnp.int32),
        pltpu.VMEM((NCH_PH, K), jnp.int32),
        pltpu.VMEM((K, DH), jnp.float32),
        pltpu.VMEM((K, DH), jnp.float32),
        pltpu.SemaphoreType.DMA,
        pltpu.SemaphoreType.DMA,
    ],
)


@jax.jit
def kernel(feat, edge_index):
    ei = edge_index.astype(jnp.int32)
    npad = EPAD - E
    src = jnp.concatenate([ei[0], jnp.zeros((npad,), jnp.int32)])
    pad_dst = N + (jnp.arange(npad, dtype=jnp.int32) % JUNK)
    dst = jnp.concatenate([ei[1], pad_dst])
    src = src.reshape(NS * CT, K)
    dst = dst.reshape(NS * CT, K)
    return _sc_gin(feat, src, dst)


# speedup vs baseline: 1.2577x; 1.0574x over previous
import jax
import jax.numpy as jnp
from jax import lax
from jax.experimental import pallas as pl
from jax.experimental.pallas import tpu as pltpu
from jax.experimental.pallas import tpu_sc as plsc

N = 10000
D = 128
DH = D // 2
E = 320000
NC = 2
NS = 16
K = 100
CT = 200
NCH_PH = 100
NPH = CT // NCH_PH
IOSUB = 10
IOROWS = N // IOSUB


def _sc_body(feat_hbm, ei_hbm, out_hbm,
             feats, acc, idx_s, idx_d, rows0, rows1, sem0, sem1):
    c = lax.axis_index("c")
    s = lax.axis_index("s")

    io_base = pl.multiple_of(s * IOROWS, 8)
    col = pl.multiple_of(c * DH, DH)

    @pl.when(s < IOSUB)
    def _init():
        pltpu.sync_copy(feat_hbm.at[pl.ds(io_base, IOROWS), pl.ds(col, DH)],
                        feats.at[pl.ds(io_base, IOROWS)])
        pltpu.sync_copy(feat_hbm.at[pl.ds(io_base, IOROWS), pl.ds(col, DH)],
                        acc.at[pl.ds(io_base, IOROWS)])

    plsc.subcore_barrier()

    for p in range(NPH):
        slab = pl.multiple_of(s * CT + p * NCH_PH, 4)
        pltpu.sync_copy(ei_hbm.at[0, pl.ds(slab, NCH_PH)], idx_s)
        pltpu.sync_copy(ei_hbm.at[1, pl.ds(slab, NCH_PH)], idx_d)

        pltpu.async_copy(feats.at[idx_s.at[0]], rows0, sem0)

        def body(i, carry):
            j0 = 2 * i
            j1 = j0 + 1
            pltpu.async_copy(feats.at[idx_s.at[j1]], rows1, sem1)
            pltpu.make_async_copy(feats.at[idx_s.at[j0]], rows0, sem0).wait()
            pltpu.sync_copy(rows0, acc.at[idx_d.at[j0]], add=True)

            @pl.when(j1 + 1 < NCH_PH)
            def _():
                pltpu.async_copy(feats.at[idx_s.at[j1 + 1]], rows0, sem0)

            pltpu.make_async_copy(feats.at[idx_s.at[j1]], rows1, sem1).wait()
            pltpu.sync_copy(rows1, acc.at[idx_d.at[j1]], add=True)
            return carry

        lax.fori_loop(0, NCH_PH // 2, body, 0)

    plsc.subcore_barrier()

    @pl.when(s < IOSUB)
    def _writeout():
        pltpu.sync_copy(acc.at[pl.ds(io_base, IOROWS)],
                        out_hbm.at[pl.ds(io_base, IOROWS), pl.ds(col, DH)])


_sc_gin = pl.kernel(
    _sc_body,
    out_type=jax.ShapeDtypeStruct((N, D), jnp.float32),
    mesh=plsc.VectorSubcoreMesh(
        core_axis_name="c", subcore_axis_name="s", num_cores=NC, num_subcores=NS
    ),
    compiler_params=pltpu.CompilerParams(use_tc_tiling_on_sc=False),
    scratch_types=[
        pltpu.VMEM_SHARED((N, DH), jnp.float32),
        pltpu.VMEM_SHARED((N, DH), jnp.float32),
        pltpu.VMEM((NCH_PH, K), jnp.int32),
        pltpu.VMEM((NCH_PH, K), jnp.int32),
        pltpu.VMEM((K, DH), jnp.float32),
        pltpu.VMEM((K, DH), jnp.float32),
        pltpu.SemaphoreType.DMA,
        pltpu.SemaphoreType.DMA,
    ],
)


@jax.jit
def kernel(feat, edge_index):
    ei = edge_index.astype(jnp.int32).reshape(2, NS * CT, K)
    return _sc_gin(feat, ei)
